# Initial kernel scaffold; baseline (speedup 1.0000x reference)
import jax
import jax.numpy as jnp
from jax.experimental import pallas as pl


def kernel(atom_idx, ring_pairs, atom_hvs, pos_hvs, closure_tag, W, b):
    raise NotImplementedError("write your pallas kernel here")


# speedup vs baseline: 28.5945x; 28.5945x over previous
import functools

import jax
import jax.numpy as jnp
from jax import lax
from jax.experimental import pallas as pl
from jax.experimental.pallas import tpu as pltpu

_B = 256
_L = 128
_D = 10000
_DP = 10240
_KB = 2048
_NK = _DP // _KB
_NA = 10
_PROJ = 256


def _dot_nt(x, y):
    return lax.dot_general(x, y, (((1,), (1,)), ((), ())),
                           preferred_element_type=jnp.float32)


def _dot_nn(x, y):
    return lax.dot_general(x, y, (((1,), (0,)), ((), ())),
                           preferred_element_type=jnp.float32)


def _tc_body(idx_ref, rp_ref, a_ref, p_ref, tag_ref, w_ref, b_ref,
             out_ref, h2_ref):
    k = pl.program_id(0)
    P = p_ref[...]
    Wk = w_ref[...]

    @pl.when(k == 0)
    def _():
        h2_ref[...] = jnp.zeros_like(h2_ref)
        out_ref[...] = jnp.broadcast_to(b_ref[0:1, :], (_B, _PROJ))

    for a in range(_NA):
        wa = Wk * a_ref[a:a + 1, :]
        h2_ref[a * _L:(a + 1) * _L, :] += _dot_nt(P, wa)

    iota_l = lax.broadcasted_iota(jnp.int32, (_B, _L), 1)
    ohi = (rp_ref[:, 0:1] == iota_l).astype(jnp.float32)
    ohj = (rp_ref[:, 1:2] == iota_l).astype(jnp.float32)
    pi = _dot_nn(ohi, P)
    pj = _dot_nn(ohj, P)
    r = pi * (pj * tag_ref[...])
    out_ref[...] += _dot_nt(r, Wk)

    @pl.when(k == _NK - 1)
    def _():
        idx = idx_ref[...]
        for a in range(_NA):
            oh = (idx == a).astype(jnp.float32)
            out_ref[...] += _dot_nn(oh, h2_ref[a * _L:(a + 1) * _L, :])


@functools.partial(jax.jit, static_argnames=())
def kernel(atom_idx, ring_pairs, atom_hvs, pos_hvs, closure_tag, W, b):
    pad = _DP - _D
    pos_p = jnp.pad(pos_hvs, ((0, 0), (0, pad)))
    atom_p = jnp.pad(atom_hvs, ((0, 16 - _NA), (0, pad)))
    w_p = jnp.pad(W, ((0, 0), (0, pad)))
    tag_p = jnp.pad(closure_tag, (0, pad)).reshape(1, _DP)
    idx = atom_idx.astype(jnp.int32)
    rp = ring_pairs.astype(jnp.int32)
    b2 = b.reshape(1, _PROJ)

    grid = (_NK,)
    out = pl.pallas_call(
        _tc_body,
        grid=grid,
        in_specs=[
            pl.BlockSpec((_B, _L), lambda k: (0, 0)),
            pl.BlockSpec((_B, 2), lambda k: (0, 0)),
            pl.BlockSpec((16, _KB), lambda k: (0, k)),
            pl.BlockSpec((_L, _KB), lambda k: (0, k)),
            pl.BlockSpec((1, _KB), lambda k: (0, k)),
            pl.BlockSpec((_PROJ, _KB), lambda k: (0, k)),
            pl.BlockSpec((1, _PROJ), lambda k: (0, 0)),
        ],
        out_specs=pl.BlockSpec((_B, _PROJ), lambda k: (0, 0)),
        out_shape=jax.ShapeDtypeStruct((_B, _PROJ), jnp.float32),
        scratch_shapes=[pltpu.VMEM((_NA * _L, _PROJ), jnp.float32)],
    )(idx, rp, atom_p, pos_p, tag_p, w_p, b2)
    return out
